# Initial kernel scaffold; baseline (speedup 1.0000x reference)
import jax
import jax.numpy as jnp
from jax.experimental import pallas as pl


def kernel(x, edge_index, W_z, b_z, W_r, b_r, W_h, b_h, lz_W, lz_b, lr_W, lr_b, lh_W, lh_b, att, W_lin, b_lin):
    raise NotImplementedError("write your pallas kernel here")


# speedup vs baseline: 17.9647x; 17.9647x over previous
import functools

import jax
import jax.numpy as jnp
from jax import lax
from jax.experimental import pallas as pl
from jax.experimental.pallas import tpu as pltpu
from jax.experimental.pallas import tpu_sc as plsc

N = 10000
E = 160000
DF = 256
DH = 32
DC = 2 * DH

NC = 2
NS = 16
NW = NC * NS
CHUNK = 128
NCHUNK = 40
EPT = CHUNK * NCHUNK
E_PAD = EPT * NW
STRIPE = 632
N_PAD = NS * STRIPE
DEG_W = 16
TRASH = N + 8

def _deg_body(dst_hbm, ones_hbm, zeros_hbm, out_hbm, dst_v, ones_v, acc_sh):
    c = lax.axis_index("c")
    s = lax.axis_index("s")
    wid = c * NS + s
    pltpu.sync_copy(dst_hbm.at[wid], dst_v)
    pltpu.sync_copy(ones_hbm, ones_v)
    pltpu.sync_copy(zeros_hbm, acc_sh.at[pl.ds(s * STRIPE, STRIPE)])
    plsc.subcore_barrier()

    def body(j, carry):
        pltpu.sync_copy(ones_v, acc_sh.at[dst_v.at[j]], add=True)
        return carry

    lax.fori_loop(0, NCHUNK, body, 0)
    plsc.subcore_barrier()
    pltpu.sync_copy(acc_sh.at[pl.ds(s * STRIPE, STRIPE)], out_hbm.at[c, s])


def _scatter_body(src_hbm, dst_hbm, y_hbm, zeros_hbm, out_hbm,
                  src_v, dst_v, rows_v, acc_sh, sem):
    c = lax.axis_index("c")
    s = lax.axis_index("s")
    wid = c * NS + s
    pltpu.sync_copy(src_hbm.at[wid], src_v)
    pltpu.sync_copy(dst_hbm.at[wid], dst_v)
    pltpu.sync_copy(zeros_hbm, acc_sh.at[pl.ds(s * STRIPE, STRIPE)])
    plsc.subcore_barrier()

    def body(j, carry):
        pltpu.async_copy(y_hbm.at[src_v.at[j]], rows_v, sem).wait()
        pltpu.sync_copy(rows_v, acc_sh.at[dst_v.at[j]], add=True)
        return carry

    lax.fori_loop(0, NCHUNK, body, 0)
    plsc.subcore_barrier()
    pltpu.sync_copy(acc_sh.at[pl.ds(s * STRIPE, STRIPE)], out_hbm.at[c, s])


@functools.cache
def _sc_kernels():
    mesh = plsc.VectorSubcoreMesh(core_axis_name="c", subcore_axis_name="s")
    deg_k = pl.kernel(
        _deg_body,
        out_type=jax.ShapeDtypeStruct((NC, NS, STRIPE, DEG_W), jnp.float32),
        mesh=mesh,
        scratch_types=[
            pltpu.VMEM((NCHUNK, CHUNK), jnp.int32),
            pltpu.VMEM((CHUNK, DEG_W), jnp.float32),
            pltpu.VMEM_SHARED((N_PAD, DEG_W), jnp.float32),
        ],
        compiler_params=pltpu.CompilerParams(use_tc_tiling_on_sc=False),
    )
    scatter_k = pl.kernel(
        _scatter_body,
        out_type=jax.ShapeDtypeStruct((NC, NS, STRIPE, DC), jnp.float32),
        mesh=mesh,
        scratch_types=[
            pltpu.VMEM((NCHUNK, CHUNK), jnp.int32),
            pltpu.VMEM((NCHUNK, CHUNK), jnp.int32),
            pltpu.VMEM((CHUNK, DC), jnp.float32),
            pltpu.VMEM_SHARED((N_PAD, DC), jnp.float32),
            pltpu.SemaphoreType.DMA,
        ],
        compiler_params=pltpu.CompilerParams(use_tc_tiling_on_sc=False),
    )
    return deg_k, scatter_k


def _xw_body(x_ref, w_ref, deg_ref, y_ref):
    xw = jnp.dot(x_ref[...], w_ref[...], preferred_element_type=jnp.float32)
    deg = deg_ref[0:N, 0:1] + deg_ref[0:N, 1:2] + 1.0
    dis = lax.rsqrt(deg)
    y_ref[...] = xw * dis


def _xw_call(x2d, w_cat, deg_t):
    return pl.pallas_call(
        _xw_body,
        out_shape=jax.ShapeDtypeStruct((N, DC), jnp.float32),
    )(x2d, w_cat, deg_t)


def _gate_body(s_ref, y_ref, deg_ref, az_ref, ah_ref, bz_ref, bh_ref,
               lzb_ref, lhb_ref, wl_ref, bl_ref, out_ref):
    ssum = s_ref[0, 0:N, :] + s_ref[1, 0:N, :]
    deg = deg_ref[0:N, 0:1] + deg_ref[0:N, 1:2] + 1.0
    dis = lax.rsqrt(deg)
    cfull = dis * (ssum + y_ref[...])
    cz = cfull[:, 0:DH]
    ch = cfull[:, DH:DC]
    a_z = az_ref[...]
    a_h = ah_ref[...]
    bz = jnp.dot(bz_ref[...], a_z, preferred_element_type=jnp.float32) + lzb_ref[...]
    bh = jnp.dot(bh_ref[...], a_h, preferred_element_type=jnp.float32) + lhb_ref[...]
    z = jax.nn.sigmoid(jnp.dot(cz, a_z, preferred_element_type=jnp.float32) + bz)
    t = jnp.tanh(jnp.dot(ch, a_h, preferred_element_type=jnp.float32) + bh)
    g = (1.0 - z) * t
    out_ref[...] = jnp.dot(g, wl_ref[...], preferred_element_type=jnp.float32) + bl_ref[...]


def _gate_call(s_part, y, deg_t, a_z, a_h, b_z, b_h, lz_b, lh_b, w_lin, b_lin):
    return pl.pallas_call(
        _gate_body,
        out_shape=jax.ShapeDtypeStruct((N, 1), jnp.float32),
    )(s_part, y, deg_t, a_z, a_h, b_z, b_h, lz_b, lh_b, w_lin, b_lin)


def kernel(x, edge_index, W_z, b_z, W_r, b_r, W_h, b_h, lz_W, lz_b, lr_W,
           lr_b, lh_W, lh_b, att, W_lin, b_lin):
    x2d = x[:, :, 0]
    w_cat = jnp.concatenate([W_z, W_h], axis=1)

    pad = E_PAD - E
    src = jnp.concatenate([edge_index[0], jnp.zeros((pad,), jnp.int32)])
    dst = jnp.concatenate([edge_index[1], jnp.full((pad,), TRASH, jnp.int32)])
    src3 = src.reshape(NW, NCHUNK, CHUNK)
    dst3 = dst.reshape(NW, NCHUNK, CHUNK)

    ones1 = jnp.ones((CHUNK, DEG_W), jnp.float32)
    zeros1 = jnp.zeros((STRIPE, DEG_W), jnp.float32)
    zeros2 = jnp.zeros((STRIPE, DC), jnp.float32)

    deg_kernel, scatter_kernel = _sc_kernels()
    deg_part = deg_kernel(dst3, ones1, zeros1)
    deg_t = deg_part.reshape(NC, N_PAD, DEG_W)[:, :, 0].T

    y = _xw_call(x2d, w_cat, deg_t)

    s_part = scatter_kernel(src3, dst3, y, zeros2).reshape(NC, N_PAD, DC)

    out2 = _gate_call(
        s_part, y, deg_t,
        lz_W[0:DH, :], lh_W[0:DH, :],
        b_z.reshape(1, DH), b_h.reshape(1, DH),
        lz_b.reshape(1, DH), lh_b.reshape(1, DH),
        W_lin, b_lin.reshape(1, 1),
    )
    return out2[:, 0]


# speedup vs baseline: 26.0133x; 1.4480x over previous
import functools

import jax
import jax.numpy as jnp
from jax import lax
from jax.experimental import pallas as pl
from jax.experimental.pallas import tpu as pltpu
from jax.experimental.pallas import tpu_sc as plsc

N = 10000
E = 160000
DF = 256
DH = 32
DC = 2 * DH

NC = 2
NS = 16
NW = NC * NS
CHUNK = 128
NCHUNK = 40
EPT = CHUNK * NCHUNK
E_PAD = EPT * NW
STRIPE = 632
N_PAD = NS * STRIPE
DEG_W = 16
TRASH = N + 8

def _deg_body(dst_hbm, ones_hbm, zeros_hbm, out_hbm, dst_v, ones_v, acc_sh):
    c = lax.axis_index("c")
    s = lax.axis_index("s")
    wid = c * NS + s
    pltpu.sync_copy(dst_hbm.at[wid], dst_v)
    pltpu.sync_copy(ones_hbm, ones_v)
    pltpu.sync_copy(zeros_hbm, acc_sh.at[pl.ds(s * STRIPE, STRIPE)])
    plsc.subcore_barrier()

    def body(j, carry):
        pltpu.sync_copy(ones_v, acc_sh.at[dst_v.at[j]], add=True)
        return carry

    lax.fori_loop(0, NCHUNK, body, 0)
    plsc.subcore_barrier()
    pltpu.sync_copy(acc_sh.at[pl.ds(s * STRIPE, STRIPE)], out_hbm.at[c, s])


RING = 4
NRING = NCHUNK // RING


def _scatter_body(src_hbm, dst_hbm, y_hbm, zeros_hbm, out_hbm,
                  src_v, dst_v, buf_a, buf_b, acc_sh,
                  gsem_a, gsem_b, ssem_a, ssem_b):
    c = lax.axis_index("c")
    s = lax.axis_index("s")
    wid = c * NS + s
    pltpu.sync_copy(src_hbm.at[wid], src_v)
    pltpu.sync_copy(dst_hbm.at[wid], dst_v)
    pltpu.sync_copy(zeros_hbm, acc_sh.at[pl.ds(s * STRIPE, STRIPE)])
    plsc.subcore_barrier()

    def fire_gathers(ring, buf, gsem):
        for b in range(RING):
            pltpu.async_copy(y_hbm.at[src_v.at[ring * RING + b]], buf.at[b], gsem)

    def drain_gathers(buf, gsem):
        for b in range(RING):
            pltpu.make_async_copy(y_hbm.at[src_v.at[0]], buf.at[b], gsem).wait()

    def fire_scatters(ring, buf, ssem):
        for b in range(RING):
            pltpu.async_copy(buf.at[b], acc_sh.at[dst_v.at[ring * RING + b]],
                             ssem, add=True)

    def drain_scatters(buf, ssem):
        for b in range(RING):
            pltpu.make_async_copy(buf.at[b], acc_sh.at[dst_v.at[0]], ssem).wait()

    fire_gathers(0, buf_a, gsem_a)

    def body(i, carry):
        ra = 2 * i
        rb = 2 * i + 1

        @pl.when(i > 0)
        def _():
            drain_scatters(buf_b, ssem_b)

        fire_gathers(rb, buf_b, gsem_b)
        drain_gathers(buf_a, gsem_a)
        fire_scatters(ra, buf_a, ssem_a)
        drain_scatters(buf_a, ssem_a)

        @pl.when(ra + 2 < NRING)
        def _():
            fire_gathers(ra + 2, buf_a, gsem_a)

        drain_gathers(buf_b, gsem_b)
        fire_scatters(rb, buf_b, ssem_b)
        return carry

    lax.fori_loop(0, NRING // 2, body, 0)
    drain_scatters(buf_b, ssem_b)
    plsc.subcore_barrier()
    pltpu.sync_copy(acc_sh.at[pl.ds(s * STRIPE, STRIPE)], out_hbm.at[c, s])


@functools.cache
def _sc_kernels():
    mesh = plsc.VectorSubcoreMesh(core_axis_name="c", subcore_axis_name="s")
    deg_k = pl.kernel(
        _deg_body,
        out_type=jax.ShapeDtypeStruct((NC, NS, STRIPE, DEG_W), jnp.float32),
        mesh=mesh,
        scratch_types=[
            pltpu.VMEM((NCHUNK, CHUNK), jnp.int32),
            pltpu.VMEM((CHUNK, DEG_W), jnp.float32),
            pltpu.VMEM_SHARED((N_PAD, DEG_W), jnp.float32),
        ],
        compiler_params=pltpu.CompilerParams(use_tc_tiling_on_sc=False),
    )
    scatter_k = pl.kernel(
        _scatter_body,
        out_type=jax.ShapeDtypeStruct((NC, NS, STRIPE, DC), jnp.float32),
        mesh=mesh,
        scratch_types=[
            pltpu.VMEM((NCHUNK, CHUNK), jnp.int32),
            pltpu.VMEM((NCHUNK, CHUNK), jnp.int32),
            pltpu.VMEM((RING, CHUNK, DC), jnp.float32),
            pltpu.VMEM((RING, CHUNK, DC), jnp.float32),
            pltpu.VMEM_SHARED((N_PAD, DC), jnp.float32),
            pltpu.SemaphoreType.DMA,
            pltpu.SemaphoreType.DMA,
            pltpu.SemaphoreType.DMA,
            pltpu.SemaphoreType.DMA,
        ],
        compiler_params=pltpu.CompilerParams(use_tc_tiling_on_sc=False),
    )
    return deg_k, scatter_k


def _xw_body(x_ref, w_ref, deg_ref, y_ref):
    xw = jnp.dot(x_ref[...], w_ref[...], preferred_element_type=jnp.float32)
    deg = deg_ref[0, 0:N, 0:1] + deg_ref[1, 0:N, 0:1] + 1.0
    dis = lax.rsqrt(deg)
    y_ref[0:N, :] = xw * dis
    y_ref[N:N_PAD, :] = jnp.zeros((N_PAD - N, DC), jnp.float32)


def _xw_call(x2d, w_cat, deg3):
    return pl.pallas_call(
        _xw_body,
        out_shape=jax.ShapeDtypeStruct((N_PAD, DC), jnp.float32),
    )(x2d, w_cat, deg3)


def _gate_body(s_ref, y_ref, deg_ref, az_ref, ah_ref, bz_ref, bh_ref,
               lzb_ref, lhb_ref, wl_ref, bl_ref, out_ref):
    ssum = s_ref[0, 0:N, :] + s_ref[1, 0:N, :]
    deg = deg_ref[0, 0:N, 0:1] + deg_ref[1, 0:N, 0:1] + 1.0
    dis = lax.rsqrt(deg)
    cfull = dis * (ssum + y_ref[0:N, :])
    cz = cfull[:, 0:DH]
    ch = cfull[:, DH:DC]
    a_z = az_ref[...]
    a_h = ah_ref[...]
    bz = jnp.dot(bz_ref[...], a_z, preferred_element_type=jnp.float32) + lzb_ref[...]
    bh = jnp.dot(bh_ref[...], a_h, preferred_element_type=jnp.float32) + lhb_ref[...]
    z = jax.nn.sigmoid(jnp.dot(cz, a_z, preferred_element_type=jnp.float32) + bz)
    t = jnp.tanh(jnp.dot(ch, a_h, preferred_element_type=jnp.float32) + bh)
    g = (1.0 - z) * t
    out_ref[...] = jnp.dot(g, wl_ref[...], preferred_element_type=jnp.float32) + bl_ref[...]


def _gate_call(s_part, y, deg_t, a_z, a_h, b_z, b_h, lz_b, lh_b, w_lin, b_lin):
    return pl.pallas_call(
        _gate_body,
        out_shape=jax.ShapeDtypeStruct((N, 1), jnp.float32),
    )(s_part, y, deg_t, a_z, a_h, b_z, b_h, lz_b, lh_b, w_lin, b_lin)


def kernel(x, edge_index, W_z, b_z, W_r, b_r, W_h, b_h, lz_W, lz_b, lr_W,
           lr_b, lh_W, lh_b, att, W_lin, b_lin):
    x2d = x[:, :, 0]
    w_cat = jnp.concatenate([W_z, W_h], axis=1)

    ei_pad = jnp.pad(edge_index, ((0, 0), (0, E_PAD - E)),
                     constant_values=TRASH)
    src3 = ei_pad[0].reshape(NW, NCHUNK, CHUNK)
    dst3 = ei_pad[1].reshape(NW, NCHUNK, CHUNK)

    ones1 = jnp.ones((CHUNK, DEG_W), jnp.float32)
    zeros1 = jnp.zeros((STRIPE, DEG_W), jnp.float32)
    zeros2 = jnp.zeros((STRIPE, DC), jnp.float32)

    deg_kernel, scatter_kernel = _sc_kernels()
    deg3 = deg_kernel(dst3, ones1, zeros1).reshape(NC, N_PAD, DEG_W)

    y = _xw_call(x2d, w_cat, deg3)

    s_part = scatter_kernel(src3, dst3, y, zeros2).reshape(NC, N_PAD, DC)

    out2 = _gate_call(
        s_part, y, deg3,
        lz_W[0:DH, :], lh_W[0:DH, :],
        b_z.reshape(1, DH), b_h.reshape(1, DH),
        lz_b.reshape(1, DH), lh_b.reshape(1, DH),
        W_lin, b_lin.reshape(1, 1),
    )
    return out2[:, 0]


# speedup vs baseline: 43.7663x; 1.6825x over previous
import functools

import jax
import jax.numpy as jnp
from jax import lax
from jax.experimental import pallas as pl
from jax.experimental.pallas import tpu as pltpu
from jax.experimental.pallas import tpu_sc as plsc

N = 10000
E = 160000
DF = 256
DH = 32
DC = 2 * DH

NC = 2
NS = 16
NW = NC * NS
CHUNK = 128
NCHUNK = 40
EPT = CHUNK * NCHUNK
E_PAD = EPT * NW
STRIPE = 632
N_PAD = NS * STRIPE
DEG_W = 16
TRASH = N + 8

def _deg_body(dst_hbm, ones_hbm, zeros_hbm, out_hbm, dst_v, ones_v, acc_sh):
    c = lax.axis_index("c")
    s = lax.axis_index("s")
    wid = c * NS + s
    pltpu.sync_copy(dst_hbm.at[wid], dst_v)
    pltpu.sync_copy(ones_hbm, ones_v)
    pltpu.sync_copy(zeros_hbm, acc_sh.at[pl.ds(s * STRIPE, STRIPE)])
    plsc.subcore_barrier()

    def body(j, carry):
        pltpu.sync_copy(ones_v, acc_sh.at[dst_v.at[j]], add=True)
        return carry

    lax.fori_loop(0, NCHUNK, body, 0)
    plsc.subcore_barrier()
    pltpu.sync_copy(acc_sh.at[pl.ds(s * STRIPE, STRIPE)], out_hbm.at[c, s])


RING = 4
NRING = NCHUNK // RING


def _scatter_body(src_hbm, dst_hbm, y_hbm, zeros_hbm, out_hbm,
                  src_v, dst_v, buf_a, buf_b, acc_sh,
                  gsem_a, gsem_b, ssem_a, ssem_b):
    c = lax.axis_index("c")
    s = lax.axis_index("s")
    wid = c * NS + s
    pltpu.sync_copy(src_hbm.at[wid], src_v)
    pltpu.sync_copy(dst_hbm.at[wid], dst_v)
    pltpu.sync_copy(zeros_hbm, acc_sh.at[pl.ds(s * STRIPE, STRIPE)])
    plsc.subcore_barrier()

    def fire_gathers(ring, buf, gsem):
        for b in range(RING):
            pltpu.async_copy(y_hbm.at[src_v.at[ring * RING + b]], buf.at[b], gsem)

    def drain_gathers(buf, gsem):
        for b in range(RING):
            pltpu.make_async_copy(y_hbm.at[src_v.at[0]], buf.at[b], gsem).wait()

    def fire_scatters(ring, buf, ssem):
        for b in range(RING):
            pltpu.async_copy(buf.at[b], acc_sh.at[dst_v.at[ring * RING + b]],
                             ssem, add=True)

    def drain_scatters(buf, ssem):
        for b in range(RING):
            pltpu.make_async_copy(buf.at[b], acc_sh.at[dst_v.at[0]], ssem).wait()

    fire_gathers(0, buf_a, gsem_a)

    def body(i, carry):
        ra = 2 * i
        rb = 2 * i + 1

        @pl.when(i > 0)
        def _():
            drain_scatters(buf_b, ssem_b)

        fire_gathers(rb, buf_b, gsem_b)
        drain_gathers(buf_a, gsem_a)
        fire_scatters(ra, buf_a, ssem_a)
        drain_scatters(buf_a, ssem_a)

        @pl.when(ra + 2 < NRING)
        def _():
            fire_gathers(ra + 2, buf_a, gsem_a)

        drain_gathers(buf_b, gsem_b)
        fire_scatters(rb, buf_b, ssem_b)
        return carry

    lax.fori_loop(0, NRING // 2, body, 0)
    drain_scatters(buf_b, ssem_b)
    plsc.subcore_barrier()
    pltpu.sync_copy(acc_sh.at[pl.ds(s * STRIPE, STRIPE)], out_hbm.at[c, s])


@functools.cache
def _sc_kernels():
    mesh = plsc.VectorSubcoreMesh(core_axis_name="c", subcore_axis_name="s")
    deg_k = pl.kernel(
        _deg_body,
        out_type=jax.ShapeDtypeStruct((NC, NS, STRIPE, DEG_W), jnp.float32),
        mesh=mesh,
        scratch_types=[
            pltpu.VMEM((NCHUNK, CHUNK), jnp.int32),
            pltpu.VMEM((CHUNK, DEG_W), jnp.float32),
            pltpu.VMEM_SHARED((N_PAD, DEG_W), jnp.float32),
        ],
        compiler_params=pltpu.CompilerParams(use_tc_tiling_on_sc=False),
    )
    scatter_k = pl.kernel(
        _scatter_body,
        out_type=jax.ShapeDtypeStruct((NC, NS, STRIPE, DC), jnp.float32),
        mesh=mesh,
        scratch_types=[
            pltpu.VMEM((NCHUNK, CHUNK), jnp.int32),
            pltpu.VMEM((NCHUNK, CHUNK), jnp.int32),
            pltpu.VMEM((RING, CHUNK, DC), jnp.float32),
            pltpu.VMEM((RING, CHUNK, DC), jnp.float32),
            pltpu.VMEM_SHARED((N_PAD, DC), jnp.float32),
            pltpu.SemaphoreType.DMA,
            pltpu.SemaphoreType.DMA,
            pltpu.SemaphoreType.DMA,
            pltpu.SemaphoreType.DMA,
        ],
        compiler_params=pltpu.CompilerParams(use_tc_tiling_on_sc=False),
    )
    return deg_k, scatter_k


def _xw_body(x_ref, w_ref, deg_ref, y_ref):
    xw = jnp.dot(x_ref[...], w_ref[...], preferred_element_type=jnp.float32)
    deg = deg_ref[0, 0:N, 0:1] + deg_ref[1, 0:N, 0:1] + 1.0
    dis = lax.rsqrt(deg)
    y_ref[0:N, :] = xw * dis
    y_ref[N:N_PAD, :] = jnp.zeros((N_PAD - N, DC), jnp.float32)


def _xw_call(x2d, w_cat, deg3):
    return pl.pallas_call(
        _xw_body,
        out_shape=jax.ShapeDtypeStruct((N_PAD, DC), jnp.float32),
    )(x2d, w_cat, deg3)


def _gate_body(s_ref, y_ref, deg_ref, az_ref, ah_ref, bz_ref, bh_ref,
               lzb_ref, lhb_ref, wl_ref, bl_ref, out_ref):
    ssum = s_ref[0, 0:N, :] + s_ref[1, 0:N, :]
    deg = deg_ref[0, 0:N, 0:1] + deg_ref[1, 0:N, 0:1] + 1.0
    dis = lax.rsqrt(deg)
    cfull = dis * (ssum + y_ref[0:N, :])
    cz = cfull[:, 0:DH]
    ch = cfull[:, DH:DC]
    a_z = az_ref[...]
    a_h = ah_ref[...]
    bz = jnp.dot(bz_ref[...], a_z, preferred_element_type=jnp.float32) + lzb_ref[...]
    bh = jnp.dot(bh_ref[...], a_h, preferred_element_type=jnp.float32) + lhb_ref[...]
    z = jax.nn.sigmoid(jnp.dot(cz, a_z, preferred_element_type=jnp.float32) + bz)
    t = jnp.tanh(jnp.dot(ch, a_h, preferred_element_type=jnp.float32) + bh)
    g = (1.0 - z) * t
    out_ref[...] = jnp.dot(g, wl_ref[...], preferred_element_type=jnp.float32) + bl_ref[...]


def _gate_call(s_part, y, deg_t, a_z, a_h, b_z, b_h, lz_b, lh_b, w_lin, b_lin):
    return pl.pallas_call(
        _gate_body,
        out_shape=jax.ShapeDtypeStruct((N, 1), jnp.float32),
    )(s_part, y, deg_t, a_z, a_h, b_z, b_h, lz_b, lh_b, w_lin, b_lin)


def kernel(x, edge_index, W_z, b_z, W_r, b_r, W_h, b_h, lz_W, lz_b, lr_W,
           lr_b, lh_W, lh_b, att, W_lin, b_lin):
    x2d = x[:, :, 0]
    w_cat = jnp.concatenate([W_z, W_h], axis=1)

    pad_idx = N + lax.iota(jnp.int32, E_PAD - E) % (N_PAD - N)
    ei_pad = jnp.concatenate([edge_index, jnp.stack([pad_idx, pad_idx])], axis=1)
    src3 = ei_pad[0].reshape(NW, NCHUNK, CHUNK)
    dst3 = ei_pad[1].reshape(NW, NCHUNK, CHUNK)

    ones1 = jnp.ones((CHUNK, DEG_W), jnp.float32)
    zeros1 = jnp.zeros((STRIPE, DEG_W), jnp.float32)
    zeros2 = jnp.zeros((STRIPE, DC), jnp.float32)

    deg_kernel, scatter_kernel = _sc_kernels()
    deg3 = deg_kernel(dst3, ones1, zeros1).reshape(NC, N_PAD, DEG_W)

    y = _xw_call(x2d, w_cat, deg3)

    s_part = scatter_kernel(src3, dst3, y, zeros2).reshape(NC, N_PAD, DC)

    out2 = _gate_call(
        s_part, y, deg3,
        lz_W[0:DH, :], lh_W[0:DH, :],
        b_z.reshape(1, DH), b_h.reshape(1, DH),
        lz_b.reshape(1, DH), lh_b.reshape(1, DH),
        W_lin, b_lin.reshape(1, 1),
    )
    return out2[:, 0]
